# baseline (speedup 1.0000x reference)
import functools

import jax
import jax.numpy as jnp
import numpy as np
from jax import lax
from jax.experimental import pallas as pl
from jax.experimental.pallas import tpu as pltpu
from jax.experimental.pallas import tpu_sc as plsc

B = 4
N = 8192
S1 = 512
NS1 = 32
S2 = 128
NS2 = 64
R1SQ = 0.2 ** 2
R2SQ = 0.4 ** 2
BN_SCALE = 1.0 / np.sqrt(1.0 + 1e-5)


def _nt(a, b):
    return lax.dot_general(a, b, (((1,), (1,)), ((), ())),
                           preferred_element_type=jnp.float32)


def _fps_body(npoint, rows, cols, xyz_ref, cx_ref, cy_ref, cz_ref):
    n = rows * cols
    xyz = xyz_ref[...]
    gidx = (lax.broadcasted_iota(jnp.int32, (B, rows, cols), 1) * cols
            + lax.broadcasted_iota(jnp.int32, (B, rows, cols), 2))
    gidx3 = ((lax.broadcasted_iota(jnp.int32, (B, 3 * rows, cols), 1) % rows)
             * cols
             + lax.broadcasted_iota(jnp.int32, (B, 3 * rows, cols), 2))
    lane_s = lax.broadcasted_iota(jnp.int32, (B, npoint), 1)

    def rmin(a):
        return jnp.min(jnp.min(a, axis=2, keepdims=True), axis=1, keepdims=True)

    def rmax(a):
        return jnp.max(jnp.max(a, axis=2, keepdims=True), axis=1, keepdims=True)

    def body(i, state):
        dists, far, ax, ay, az = state
        sel = gidx3 == far
        csum = jnp.sum(jnp.where(sel, xyz, 0.0), axis=2, keepdims=True)
        cx = jnp.sum(csum[:, 0:rows], axis=1, keepdims=True)
        cy = jnp.sum(csum[:, rows:2 * rows], axis=1, keepdims=True)
        cz = jnp.sum(csum[:, 2 * rows:], axis=1, keepdims=True)
        hit = lane_s == i
        ax = jnp.where(hit, cx[:, 0], ax)
        ay = jnp.where(hit, cy[:, 0], ay)
        az = jnp.where(hit, cz[:, 0], az)
        cb = jnp.concatenate([
            jnp.broadcast_to(cx, (B, rows, 1)),
            jnp.broadcast_to(cy, (B, rows, 1)),
            jnp.broadcast_to(cz, (B, rows, 1))], axis=1)
        dall = (xyz - cb) ** 2
        d = (dall[:, 0:rows] + dall[:, rows:2 * rows]) + dall[:, 2 * rows:]
        dists = jnp.minimum(dists, d)
        m = rmax(dists)
        far = rmin(jnp.where(dists == m, gidx, n))
        return dists, far, ax, ay, az

    dists0 = jnp.full((B, rows, cols), 1e10, dtype=jnp.float32)
    far0 = jnp.zeros((B, 1, 1), dtype=jnp.int32)
    acc0 = jnp.zeros((B, npoint), dtype=jnp.float32)
    _, _, ax, ay, az = lax.fori_loop(0, npoint, body, (dists0, far0, acc0, acc0, acc0))
    cx_ref[...] = ax
    cy_ref[...] = ay
    cz_ref[...] = az


def _fps(x, y, z, npoint, rows):
    n = x.shape[1]
    cols = n // rows
    out = jax.ShapeDtypeStruct((B, npoint), jnp.float32)
    xyz = jnp.concatenate(
        [x.reshape(B, rows, cols), y.reshape(B, rows, cols),
         z.reshape(B, rows, cols)], axis=1)
    return pl.pallas_call(
        functools.partial(_fps_body, npoint, rows, cols),
        out_shape=(out, out, out),
    )(xyz)


def _bq_body(n, nsample, r2, sc, x_ref, y_ref, z_ref, cx_ref, cy_ref, cz_ref, out_ref):
    b = pl.program_id(0)
    x = x_ref[0]
    y = y_ref[0]
    z = z_ref[0]
    bcol = lax.broadcasted_iota(jnp.int32, (sc, B), 1)

    def col(ref):
        return jnp.sum(jnp.where(bcol == b, ref[...], 0.0), axis=1, keepdims=True)

    cx = col(cx_ref)
    cy = col(cy_ref)
    cz = col(cz_ref)
    d = (cx - x) ** 2 + (cy - y) ** 2 + (cz - z) ** 2
    mask = d < r2
    mi = mask.astype(jnp.int32)
    s = mi
    sh = 1
    while sh < n:
        zpad = jnp.zeros((sc, sh), dtype=jnp.int32)
        s = s + jnp.concatenate([zpad, s[:, : n - sh]], axis=1)
        sh *= 2
    count = s[:, n - 1:]
    cols = []
    for t in range(nsample // 2):
        e = jnp.where(s <= 2 * t, 16385, jnp.where(s <= 2 * t + 1, 16384, 0))
        v = jnp.sum(e, axis=1, keepdims=True)
        cols.append(jnp.bitwise_and(v, 16383))
        cols.append(jnp.right_shift(v, 14))
    vals = jnp.concatenate(cols, axis=1)
    first = vals[:, 0:1]
    karr = lax.broadcasted_iota(jnp.int32, (sc, nsample), 1)
    out_ref[0] = jnp.where(karr < count, vals, first)


def _ball_query(x, y, z, cxT, cyT, czT, nsample, r2, sc):
    n = x.shape[1]
    s_tot = cxT.shape[0]
    grid = (B, s_tot // sc)
    return pl.pallas_call(
        functools.partial(_bq_body, n, nsample, r2, sc),
        grid=grid,
        in_specs=[
            pl.BlockSpec((1, 1, n), lambda b, j: (b, 0, 0)),
            pl.BlockSpec((1, 1, n), lambda b, j: (b, 0, 0)),
            pl.BlockSpec((1, 1, n), lambda b, j: (b, 0, 0)),
            pl.BlockSpec((sc, B), lambda b, j: (j, 0)),
            pl.BlockSpec((sc, B), lambda b, j: (j, 0)),
            pl.BlockSpec((sc, B), lambda b, j: (j, 0)),
        ],
        out_specs=pl.BlockSpec((1, sc, nsample), lambda b, j: (b, j, 0)),
        out_shape=jax.ShapeDtypeStruct((B, s_tot, nsample), jnp.int32),
    )(x[:, None, :], y[:, None, :], z[:, None, :], cxT, cyT, czT)


def _gather_rows(table, idx, D):
    ntot = idx.shape[0]
    NW = 32
    per_w = ntot // NW
    nchunk = per_w // 128
    mesh = plsc.VectorSubcoreMesh(core_axis_name="c", subcore_axis_name="s")
    idx2d = idx.reshape(NW * nchunk, 128)
    small = per_w * D * 4 <= 400 * 1024

    if small:
        scratch = [
            pltpu.VMEM((nchunk, 128), jnp.int32),
            pltpu.VMEM((per_w, D), jnp.float32),
            pltpu.SemaphoreType.DMA,
        ]

        def body(table_hbm, idx_hbm, out_hbm, idx_v, rows_v, sem):
            wid = lax.axis_index("s") * 2 + lax.axis_index("c")
            pltpu.sync_copy(idx_hbm.at[pl.ds(wid * nchunk, nchunk)], idx_v)
            copies = [
                pltpu.async_copy(
                    table_hbm.at[idx_v.at[j]],
                    rows_v.at[pl.ds(j * 128, 128)], sem)
                for j in range(nchunk)
            ]
            for c in copies:
                c.wait()
            pltpu.sync_copy(rows_v, out_hbm.at[pl.ds(wid * per_w, per_w)])
    else:
        scratch = [
            pltpu.VMEM((nchunk, 128), jnp.int32),
            pltpu.VMEM((2, 128, D), jnp.float32),
            pltpu.SemaphoreType.DMA,
            pltpu.SemaphoreType.DMA,
        ]

        def body(table_hbm, idx_hbm, out_hbm, idx_v, rows_v, sem0, sem1):
            wid = lax.axis_index("s") * 2 + lax.axis_index("c")
            sems = (sem0, sem1)
            pltpu.sync_copy(idx_hbm.at[pl.ds(wid * nchunk, nchunk)], idx_v)
            pending = pltpu.async_copy(
                table_hbm.at[idx_v.at[0]], rows_v.at[0], sems[0])
            for j in range(nchunk):
                if j + 1 < nchunk:
                    nxt = pltpu.async_copy(
                        table_hbm.at[idx_v.at[j + 1]],
                        rows_v.at[(j + 1) % 2], sems[(j + 1) % 2])
                pending.wait()
                pltpu.sync_copy(
                    rows_v.at[j % 2],
                    out_hbm.at[pl.ds(wid * per_w + j * 128, 128)])
                if j + 1 < nchunk:
                    pending = nxt

    k = pl.kernel(
        body,
        mesh=mesh,
        out_type=jax.ShapeDtypeStruct((ntot, D), jnp.float32),
        scratch_types=scratch,
        compiler_params=pltpu.CompilerParams(use_tc_tiling_on_sc=False),
    )
    return k(table, idx2d)


def _sa_body(S, ns, ng, *refs):
    g_refs = refs[:ng]
    w1_refs = refs[ng:2 * ng]
    (cexp_ref, w1x_ref, b1_ref, w2_ref, b2_ref, w3_ref, b3_ref, out_ref) = refs[2 * ng:]
    cexp = cexp_ref[0]
    h = jnp.dot(g_refs[0][0], w1_refs[0][...], preferred_element_type=jnp.float32)
    for i in range(1, ng):
        h = h + jnp.dot(g_refs[i][0], w1_refs[i][...],
                        preferred_element_type=jnp.float32)
    h = h - jnp.dot(cexp, w1x_ref[...], preferred_element_type=jnp.float32)
    h = jax.nn.relu((h + b1_ref[...]) * BN_SCALE)
    h = jnp.dot(h, w2_ref[...], preferred_element_type=jnp.float32)
    h = jax.nn.relu((h + b2_ref[...]) * BN_SCALE)
    h = jnp.dot(h, w3_ref[...], preferred_element_type=jnp.float32)
    h = jax.nn.relu((h + b3_ref[...]) * BN_SCALE)
    m = h[0:S]
    for k in range(1, ns):
        m = jnp.maximum(m, h[k * S:(k + 1) * S])
    out_ref[0] = m


def _sa_mlp_max(gws, cexp, w1x, b1, w2, b2, w3, b3, S, ns):
    ng = len(gws)
    cout = w3.shape[1]
    c1 = w2.shape[0]
    c2 = w2.shape[1]
    gspecs = [pl.BlockSpec((1, S * ns, g.shape[2]), lambda b: (b, 0, 0))
              for g, _ in gws]
    wspecs = [pl.BlockSpec((w.shape[0], c1), lambda b: (0, 0)) for _, w in gws]
    return pl.pallas_call(
        functools.partial(_sa_body, S, ns, ng),
        grid=(B,),
        in_specs=gspecs + wspecs + [
            pl.BlockSpec((1, S * ns, 3), lambda b: (b, 0, 0)),
            pl.BlockSpec((3, c1), lambda b: (0, 0)),
            pl.BlockSpec((1, c1), lambda b: (0, 0)),
            pl.BlockSpec((c1, c2), lambda b: (0, 0)),
            pl.BlockSpec((1, c2), lambda b: (0, 0)),
            pl.BlockSpec((c2, cout), lambda b: (0, 0)),
            pl.BlockSpec((1, cout), lambda b: (0, 0)),
        ],
        out_specs=pl.BlockSpec((1, S, cout), lambda b: (b, 0, 0)),
        out_shape=jax.ShapeDtypeStruct((B, S, cout), jnp.float32),
    )(*[g for g, _ in gws], *[w for _, w in gws],
      cexp, w1x, b1, w2, b2, w3, b3)


def _sa3_body(xyz_ref, f2_ref, w1x_ref, w1f_ref, b1_ref, w2_ref, b2_ref,
              w3_ref, b3_ref, out_ref):
    h = jnp.dot(xyz_ref[0], w1x_ref[...], preferred_element_type=jnp.float32)
    h = h + jnp.dot(f2_ref[0], w1f_ref[...], preferred_element_type=jnp.float32)
    h = jax.nn.relu((h + b1_ref[...]) * BN_SCALE)
    h = jnp.dot(h, w2_ref[...], preferred_element_type=jnp.float32)
    h = jax.nn.relu((h + b2_ref[...]) * BN_SCALE)
    h = jnp.dot(h, w3_ref[...], preferred_element_type=jnp.float32)
    h = jax.nn.relu((h + b3_ref[...]) * BN_SCALE)
    out_ref[0] = jnp.max(h, axis=0, keepdims=True)


def _sa3(xyz2_rows, f2, w1x, w1f, b1, w2, b2, w3, b3):
    return pl.pallas_call(
        _sa3_body,
        grid=(B,),
        in_specs=[
            pl.BlockSpec((1, S2, 3), lambda b: (b, 0, 0)),
            pl.BlockSpec((1, S2, 256), lambda b: (b, 0, 0)),
            pl.BlockSpec((3, 256), lambda b: (0, 0)),
            pl.BlockSpec((256, 256), lambda b: (0, 0)),
            pl.BlockSpec((1, 256), lambda b: (0, 0)),
            pl.BlockSpec((256, 512), lambda b: (0, 0)),
            pl.BlockSpec((1, 512), lambda b: (0, 0)),
            pl.BlockSpec((512, 1024), lambda b: (0, 0)),
            pl.BlockSpec((1, 1024), lambda b: (0, 0)),
        ],
        out_specs=pl.BlockSpec((1, 1, 1024), lambda b: (b, 0, 0)),
        out_shape=jax.ShapeDtypeStruct((B, 1, 1024), jnp.float32),
    )(xyz2_rows, f2, w1x, w1f, b1, w2, b2, w3, b3)


def _fp2_body(f3_ref, f2_ref, w1i_ref, w1f_ref, b1_ref, w2_ref, b2_ref, out_ref):
    c3 = _nt(w1i_ref[...], f3_ref[0])
    h = c3 + _nt(w1f_ref[...], f2_ref[0])
    h = jax.nn.relu((h + b1_ref[...]) * BN_SCALE)
    h = jnp.dot(w2_ref[...], h, preferred_element_type=jnp.float32)
    out_ref[0] = jax.nn.relu((h + b2_ref[...]) * BN_SCALE)


def _fp2(f3, f2, w1i, w1f, b1col, w2, b2col):
    return pl.pallas_call(
        _fp2_body,
        grid=(B,),
        in_specs=[
            pl.BlockSpec((1, 1, 1024), lambda b: (b, 0, 0)),
            pl.BlockSpec((1, S2, 256), lambda b: (b, 0, 0)),
            pl.BlockSpec((1024, 1024), lambda b: (0, 0)),
            pl.BlockSpec((1024, 256), lambda b: (0, 0)),
            pl.BlockSpec((1024, 1), lambda b: (0, 0)),
            pl.BlockSpec((1024, 1024), lambda b: (0, 0)),
            pl.BlockSpec((1024, 1), lambda b: (0, 0)),
        ],
        out_specs=pl.BlockSpec((1, 1024, S2), lambda b: (b, 0, 0)),
        out_shape=jax.ShapeDtypeStruct((B, 1024, S2), jnp.float32),
    )(f3, f2, w1i, w1f, b1col, w2, b2col)


def _fp_body(m, kx_ref, ky_ref, kz_ref, ux_ref, uy_ref, uz_ref,
             kft_ref, uft_ref, w1i_ref, w1f_ref, b1_ref, w2_ref, b2_ref, out_ref):
    b = pl.program_id(0)
    bcol = lax.broadcasted_iota(jnp.int32, (m, B), 1)

    def col(ref):
        return jnp.sum(jnp.where(bcol == b, ref[...], 0.0), axis=1, keepdims=True)

    kx = col(kx_ref)
    ky = col(ky_ref)
    kz = col(kz_ref)
    ux = ux_ref[0]
    uy = uy_ref[0]
    uz = uz_ref[0]
    d2 = (ux - kx) ** 2 + (uy - ky) ** 2 + (uz - kz) ** 2
    sub = lax.broadcasted_iota(jnp.int32, d2.shape, 0)
    at = jnp.zeros(d2.shape, dtype=jnp.float32)
    picks = []
    dd = d2
    for _ in range(3):
        mn = jnp.min(dd, axis=0, keepdims=True)
        j = jnp.min(jnp.where(dd == mn, sub, m), axis=0, keepdims=True)
        picks.append((1.0 / (mn + 1e-8), j))
        dd = jnp.where(sub == j, jnp.float32(np.inf), dd)
    norm = picks[0][0] + picks[1][0] + picks[2][0]
    for r, j in picks:
        at = at + jnp.where(sub == j, r / norm, 0.0)
    interp = jnp.dot(kft_ref[0], at, preferred_element_type=jnp.float32)
    h = jnp.dot(w1i_ref[...], interp, preferred_element_type=jnp.float32)
    h = h + _nt(w1f_ref[...], uft_ref[0])
    h = jax.nn.relu((h + b1_ref[...]) * BN_SCALE)
    h = jnp.dot(w2_ref[...], h, preferred_element_type=jnp.float32)
    out_ref[0] = jax.nn.relu((h + b2_ref[...]) * BN_SCALE)


def _fp(kxT, kyT, kzT, ux, uy, uz, kfT, uf, w1i, w1f, b1col, w2, b2col, nchunk):
    m = kxT.shape[0]
    ntot = ux.shape[1]
    cin = kfT.shape[1]
    cf = w1f.shape[1]
    cout = w2.shape[0]
    cmid = w2.shape[1]
    grid = (B, ntot // nchunk)
    return pl.pallas_call(
        functools.partial(_fp_body, m),
        grid=grid,
        in_specs=[
            pl.BlockSpec((m, B), lambda b, j: (0, 0)),
            pl.BlockSpec((m, B), lambda b, j: (0, 0)),
            pl.BlockSpec((m, B), lambda b, j: (0, 0)),
            pl.BlockSpec((1, 1, nchunk), lambda b, j: (b, 0, j)),
            pl.BlockSpec((1, 1, nchunk), lambda b, j: (b, 0, j)),
            pl.BlockSpec((1, 1, nchunk), lambda b, j: (b, 0, j)),
            pl.BlockSpec((1, cin, m), lambda b, j: (b, 0, 0)),
            pl.BlockSpec((1, nchunk, cf), lambda b, j: (b, j, 0)),
            pl.BlockSpec((cmid, cin), lambda b, j: (0, 0)),
            pl.BlockSpec((cmid, cf), lambda b, j: (0, 0)),
            pl.BlockSpec((cmid, 1), lambda b, j: (0, 0)),
            pl.BlockSpec((cout, cmid), lambda b, j: (0, 0)),
            pl.BlockSpec((cout, 1), lambda b, j: (0, 0)),
        ],
        out_specs=pl.BlockSpec((1, cout, nchunk), lambda b, j: (b, 0, j)),
        out_shape=jax.ShapeDtypeStruct((B, cout, ntot), jnp.float32),
    )(kxT, kyT, kzT, ux[:, None, :], uy[:, None, :], uz[:, None, :],
      kfT, uf, w1i, w1f, b1col, w2, b2col)


def kernel(pointcloud, params):
    pc = pointcloud
    x = pc[:, :, 0]
    y = pc[:, :, 1]
    z = pc[:, :, 2]

    cx1, cy1, cz1 = _fps(x, y, z, S1, 16)
    cx1T = jnp.transpose(cx1)
    cy1T = jnp.transpose(cy1)
    cz1T = jnp.transpose(cz1)
    idx1 = _ball_query(x, y, z, cx1T, cy1T, cz1T, NS1, R1SQ, 256)

    cx2, cy2, cz2 = _fps(cx1, cy1, cz1, S2, 4)
    cx2T = jnp.transpose(cx2)
    cy2T = jnp.transpose(cy2)
    cz2T = jnp.transpose(cz2)
    idx2 = _ball_query(cx1, cy1, cz1, cx2T, cy2T, cz2T, NS2, R2SQ, S2)

    table1 = jnp.pad(pc, ((0, 0), (0, 0), (0, 10))).reshape(B * N, 16)
    boff1 = (jnp.arange(B, dtype=jnp.int32) * N)[:, None, None]
    flat1 = jnp.transpose(idx1 + boff1, (0, 2, 1)).reshape(-1)
    g1 = _gather_rows(table1, flat1, 16).reshape(B, NS1 * S1, 16)

    c1rows = jnp.stack([cx1, cy1, cz1], axis=2)
    cexp1 = jnp.tile(c1rows, (1, NS1, 1))

    p = params["sa1"]
    w1t = jnp.transpose(p[0]["W"])
    w1p = jnp.pad(w1t, ((0, 10), (0, 0)))
    f1 = _sa_mlp_max(
        [(g1, w1p)], cexp1, w1t[0:3], p[0]["b"][None],
        jnp.transpose(p[1]["W"]), p[1]["b"][None],
        jnp.transpose(p[2]["W"]), p[2]["b"][None], S1, NS1)

    boff2 = (jnp.arange(B, dtype=jnp.int32) * S1)[:, None, None]
    flat2 = jnp.transpose(idx2 + boff2, (0, 2, 1)).reshape(-1)
    table2x = jnp.pad(c1rows, ((0, 0), (0, 0), (0, 13))).reshape(B * S1, 16)
    g2x = _gather_rows(table2x, flat2, 16).reshape(B, NS2 * S2, 16)
    table2f = f1.reshape(B * S1, 128)
    g2f = _gather_rows(table2f, flat2, 128).reshape(B, NS2 * S2, 128)

    c2rows = jnp.stack([cx2, cy2, cz2], axis=2)
    cexp2 = jnp.tile(c2rows, (1, NS2, 1))

    p = params["sa2"]
    w1t2 = jnp.transpose(p[0]["W"])
    w1x2p = jnp.pad(w1t2[0:3], ((0, 13), (0, 0)))
    f2 = _sa_mlp_max(
        [(g2x, w1x2p), (g2f, w1t2[3:131])], cexp2, w1t2[0:3], p[0]["b"][None],
        jnp.transpose(p[1]["W"]), p[1]["b"][None],
        jnp.transpose(p[2]["W"]), p[2]["b"][None], S2, NS2)

    p = params["sa3"]
    f3 = _sa3(
        c2rows, f2, jnp.transpose(p[0]["W"])[0:3],
        jnp.transpose(p[0]["W"])[3:259], p[0]["b"][None],
        jnp.transpose(p[1]["W"]), p[1]["b"][None],
        jnp.transpose(p[2]["W"]), p[2]["b"][None])

    p = params["fp2"]
    fp2_out = _fp2(
        f3, f2,
        p[0]["W"][:, :1024], p[0]["W"][:, 1024:], p[0]["b"][:, None],
        p[1]["W"], p[1]["b"][:, None])

    p = params["fp1"]
    fp1_out = _fp(
        cx2T, cy2T, cz2T, cx1, cy1, cz1, fp2_out, f1,
        p[0]["W"][:, :1024], p[0]["W"][:, 1024:], p[0]["b"][:, None],
        p[1]["W"], p[1]["b"][:, None], S1)

    p = params["fp0"]
    out = _fp(
        cx1T, cy1T, cz1T, x, y, z, fp1_out, pc[:, :, 3:6],
        p[0]["W"][:, :1024], p[0]["W"][:, 1024:], p[0]["b"][:, None],
        p[1]["W"], p[1]["b"][:, None], 2048)
    return out


# speedup vs baseline: 1.0710x; 1.0710x over previous
import functools

import jax
import jax.numpy as jnp
import numpy as np
from jax import lax
from jax.experimental import pallas as pl
from jax.experimental.pallas import tpu as pltpu
from jax.experimental.pallas import tpu_sc as plsc

B = 4
N = 8192
S1 = 512
NS1 = 32
S2 = 128
NS2 = 64
R1SQ = 0.2 ** 2
R2SQ = 0.4 ** 2
BN_SCALE = 1.0 / np.sqrt(1.0 + 1e-5)


def _nt(a, b):
    return lax.dot_general(a, b, (((1,), (1,)), ((), ())),
                           preferred_element_type=jnp.float32)


def _fps_body(npoint, rows, cols, x_ref, y_ref, z_ref, cx_ref, cy_ref, cz_ref):
    n = rows * cols
    x = x_ref[...]
    y = y_ref[...]
    z = z_ref[...]
    gidx = (lax.broadcasted_iota(jnp.int32, (B, rows, cols), 1) * cols
            + lax.broadcasted_iota(jnp.int32, (B, rows, cols), 2))
    lane_s = lax.broadcasted_iota(jnp.int32, (B, npoint), 1)

    def rmin(a):
        return jnp.min(jnp.min(a, axis=2, keepdims=True), axis=1, keepdims=True)

    def rmax(a):
        return jnp.max(jnp.max(a, axis=2, keepdims=True), axis=1, keepdims=True)

    def rsum(a):
        return jnp.sum(jnp.sum(a, axis=2, keepdims=True), axis=1, keepdims=True)

    def body(i, state):
        dists, far, ax, ay, az = state
        sel = gidx == far
        cx = rsum(jnp.where(sel, x, 0.0))
        cy = rsum(jnp.where(sel, y, 0.0))
        cz = rsum(jnp.where(sel, z, 0.0))
        hit = lane_s == i
        ax = jnp.where(hit, cx[:, 0], ax)
        ay = jnp.where(hit, cy[:, 0], ay)
        az = jnp.where(hit, cz[:, 0], az)
        d = (x - cx) ** 2 + (y - cy) ** 2 + (z - cz) ** 2
        dists = jnp.minimum(dists, d)
        m = rmax(dists)
        far = rmin(jnp.where(dists == m, gidx, n))
        return dists, far, ax, ay, az

    dists0 = jnp.full((B, rows, cols), 1e10, dtype=jnp.float32)
    far0 = jnp.zeros((B, 1, 1), dtype=jnp.int32)
    acc0 = jnp.zeros((B, npoint), dtype=jnp.float32)
    _, _, ax, ay, az = lax.fori_loop(0, npoint, body, (dists0, far0, acc0, acc0, acc0))
    cx_ref[...] = ax
    cy_ref[...] = ay
    cz_ref[...] = az


def _fps(x, y, z, npoint, rows):
    n = x.shape[1]
    cols = n // rows
    out = jax.ShapeDtypeStruct((B, npoint), jnp.float32)
    r3 = lambda a: a.reshape(B, rows, cols)
    return pl.pallas_call(
        functools.partial(_fps_body, npoint, rows, cols),
        out_shape=(out, out, out),
    )(r3(x), r3(y), r3(z))


def _bq_body(n, nsample, r2, sc, x_ref, y_ref, z_ref, cx_ref, cy_ref, cz_ref, out_ref):
    b = pl.program_id(0)
    x = x_ref[0]
    y = y_ref[0]
    z = z_ref[0]
    bcol = lax.broadcasted_iota(jnp.int32, (sc, B), 1)

    def col(ref):
        return jnp.sum(jnp.where(bcol == b, ref[...], 0.0), axis=1, keepdims=True)

    cx = col(cx_ref)
    cy = col(cy_ref)
    cz = col(cz_ref)
    d = (cx - x) ** 2 + (cy - y) ** 2 + (cz - z) ** 2
    mask = d < r2
    mi = mask.astype(jnp.int32)
    s = mi
    sh = 1
    while sh < n:
        zpad = jnp.zeros((sc, sh), dtype=jnp.int32)
        s = s + jnp.concatenate([zpad, s[:, : n - sh]], axis=1)
        sh *= 2
    count = s[:, n - 1:]
    cols = []
    for t in range(nsample // 2):
        e = jnp.where(s <= 2 * t, 16385, jnp.where(s <= 2 * t + 1, 16384, 0))
        v = jnp.sum(e, axis=1, keepdims=True)
        cols.append(jnp.bitwise_and(v, 16383))
        cols.append(jnp.right_shift(v, 14))
    vals = jnp.concatenate(cols, axis=1)
    first = vals[:, 0:1]
    karr = lax.broadcasted_iota(jnp.int32, (sc, nsample), 1)
    out_ref[0] = jnp.where(karr < count, vals, first)


def _ball_query(x, y, z, cxT, cyT, czT, nsample, r2, sc):
    n = x.shape[1]
    s_tot = cxT.shape[0]
    grid = (B, s_tot // sc)
    return pl.pallas_call(
        functools.partial(_bq_body, n, nsample, r2, sc),
        grid=grid,
        in_specs=[
            pl.BlockSpec((1, 1, n), lambda b, j: (b, 0, 0)),
            pl.BlockSpec((1, 1, n), lambda b, j: (b, 0, 0)),
            pl.BlockSpec((1, 1, n), lambda b, j: (b, 0, 0)),
            pl.BlockSpec((sc, B), lambda b, j: (j, 0)),
            pl.BlockSpec((sc, B), lambda b, j: (j, 0)),
            pl.BlockSpec((sc, B), lambda b, j: (j, 0)),
        ],
        out_specs=pl.BlockSpec((1, sc, nsample), lambda b, j: (b, j, 0)),
        out_shape=jax.ShapeDtypeStruct((B, s_tot, nsample), jnp.int32),
    )(x[:, None, :], y[:, None, :], z[:, None, :], cxT, cyT, czT)


def _gather_rows(table, idx, D):
    ntot = idx.shape[0]
    NW = 32
    per_w = ntot // NW
    nchunk = per_w // 128
    mesh = plsc.VectorSubcoreMesh(core_axis_name="c", subcore_axis_name="s")
    idx2d = idx.reshape(NW * nchunk, 128)
    small = per_w * D * 4 <= 400 * 1024

    if small:
        scratch = [
            pltpu.VMEM((nchunk, 128), jnp.int32),
            pltpu.VMEM((per_w, D), jnp.float32),
            pltpu.SemaphoreType.DMA,
        ]

        def body(table_hbm, idx_hbm, out_hbm, idx_v, rows_v, sem):
            wid = lax.axis_index("s") * 2 + lax.axis_index("c")
            pltpu.sync_copy(idx_hbm.at[pl.ds(wid * nchunk, nchunk)], idx_v)
            copies = [
                pltpu.async_copy(
                    table_hbm.at[idx_v.at[j]],
                    rows_v.at[pl.ds(j * 128, 128)], sem)
                for j in range(nchunk)
            ]
            for c in copies:
                c.wait()
            pltpu.sync_copy(rows_v, out_hbm.at[pl.ds(wid * per_w, per_w)])
    else:
        scratch = [
            pltpu.VMEM((nchunk, 128), jnp.int32),
            pltpu.VMEM((2, 128, D), jnp.float32),
            pltpu.SemaphoreType.DMA,
            pltpu.SemaphoreType.DMA,
        ]

        def body(table_hbm, idx_hbm, out_hbm, idx_v, rows_v, sem0, sem1):
            wid = lax.axis_index("s") * 2 + lax.axis_index("c")
            sems = (sem0, sem1)
            pltpu.sync_copy(idx_hbm.at[pl.ds(wid * nchunk, nchunk)], idx_v)
            pending = pltpu.async_copy(
                table_hbm.at[idx_v.at[0]], rows_v.at[0], sems[0])
            for j in range(nchunk):
                if j + 1 < nchunk:
                    nxt = pltpu.async_copy(
                        table_hbm.at[idx_v.at[j + 1]],
                        rows_v.at[(j + 1) % 2], sems[(j + 1) % 2])
                pending.wait()
                pltpu.sync_copy(
                    rows_v.at[j % 2],
                    out_hbm.at[pl.ds(wid * per_w + j * 128, 128)])
                if j + 1 < nchunk:
                    pending = nxt

    k = pl.kernel(
        body,
        mesh=mesh,
        out_type=jax.ShapeDtypeStruct((ntot, D), jnp.float32),
        scratch_types=scratch,
        compiler_params=pltpu.CompilerParams(use_tc_tiling_on_sc=False),
    )
    return k(table, idx2d)


def _sa_body(S, ns, ng, *refs):
    g_refs = refs[:ng]
    w1_refs = refs[ng:2 * ng]
    (c_ref, w1x_ref, b1_ref, w2_ref, b2_ref, w3_ref, b3_ref, out_ref) = refs[2 * ng:]
    h = jnp.dot(g_refs[0][0], w1_refs[0][...], preferred_element_type=jnp.float32)
    for i in range(1, ng):
        h = h + jnp.dot(g_refs[i][0], w1_refs[i][...],
                        preferred_element_type=jnp.float32)
    o = jnp.dot(c_ref[0], w1x_ref[...], preferred_element_type=jnp.float32)
    h = jnp.concatenate(
        [h[k * S:(k + 1) * S] - o for k in range(ns)], axis=0)
    h = jax.nn.relu((h + b1_ref[...]) * BN_SCALE)
    h = jnp.dot(h, w2_ref[...], preferred_element_type=jnp.float32)
    h = jax.nn.relu((h + b2_ref[...]) * BN_SCALE)
    h = jnp.dot(h, w3_ref[...], preferred_element_type=jnp.float32)
    h = jax.nn.relu((h + b3_ref[...]) * BN_SCALE)
    m = h[0:S]
    for k in range(1, ns):
        m = jnp.maximum(m, h[k * S:(k + 1) * S])
    out_ref[0] = m


def _sa_mlp_max(gws, crows, w1x, b1, w2, b2, w3, b3, S, ns):
    ng = len(gws)
    cout = w3.shape[1]
    c1 = w2.shape[0]
    c2 = w2.shape[1]
    gspecs = [pl.BlockSpec((1, S * ns, g.shape[2]), lambda b: (b, 0, 0))
              for g, _ in gws]
    wspecs = [pl.BlockSpec((w.shape[0], c1), lambda b: (0, 0)) for _, w in gws]
    return pl.pallas_call(
        functools.partial(_sa_body, S, ns, ng),
        grid=(B,),
        in_specs=gspecs + wspecs + [
            pl.BlockSpec((1, S, 3), lambda b: (b, 0, 0)),
            pl.BlockSpec((3, c1), lambda b: (0, 0)),
            pl.BlockSpec((1, c1), lambda b: (0, 0)),
            pl.BlockSpec((c1, c2), lambda b: (0, 0)),
            pl.BlockSpec((1, c2), lambda b: (0, 0)),
            pl.BlockSpec((c2, cout), lambda b: (0, 0)),
            pl.BlockSpec((1, cout), lambda b: (0, 0)),
        ],
        out_specs=pl.BlockSpec((1, S, cout), lambda b: (b, 0, 0)),
        out_shape=jax.ShapeDtypeStruct((B, S, cout), jnp.float32),
    )(*[g for g, _ in gws], *[w for _, w in gws],
      crows, w1x, b1, w2, b2, w3, b3)


def _sa3_body(xyz_ref, f2_ref, w1x_ref, w1f_ref, b1_ref, w2_ref, b2_ref,
              w3_ref, b3_ref, out_ref):
    h = jnp.dot(xyz_ref[0], w1x_ref[...], preferred_element_type=jnp.float32)
    h = h + jnp.dot(f2_ref[0], w1f_ref[...], preferred_element_type=jnp.float32)
    h = jax.nn.relu((h + b1_ref[...]) * BN_SCALE)
    h = jnp.dot(h, w2_ref[...], preferred_element_type=jnp.float32)
    h = jax.nn.relu((h + b2_ref[...]) * BN_SCALE)
    h = jnp.dot(h, w3_ref[...], preferred_element_type=jnp.float32)
    h = jax.nn.relu((h + b3_ref[...]) * BN_SCALE)
    out_ref[0] = jnp.max(h, axis=0, keepdims=True)


def _sa3(xyz2_rows, f2, w1x, w1f, b1, w2, b2, w3, b3):
    return pl.pallas_call(
        _sa3_body,
        grid=(B,),
        in_specs=[
            pl.BlockSpec((1, S2, 3), lambda b: (b, 0, 0)),
            pl.BlockSpec((1, S2, 256), lambda b: (b, 0, 0)),
            pl.BlockSpec((3, 256), lambda b: (0, 0)),
            pl.BlockSpec((256, 256), lambda b: (0, 0)),
            pl.BlockSpec((1, 256), lambda b: (0, 0)),
            pl.BlockSpec((256, 512), lambda b: (0, 0)),
            pl.BlockSpec((1, 512), lambda b: (0, 0)),
            pl.BlockSpec((512, 1024), lambda b: (0, 0)),
            pl.BlockSpec((1, 1024), lambda b: (0, 0)),
        ],
        out_specs=pl.BlockSpec((1, 1, 1024), lambda b: (b, 0, 0)),
        out_shape=jax.ShapeDtypeStruct((B, 1, 1024), jnp.float32),
    )(xyz2_rows, f2, w1x, w1f, b1, w2, b2, w3, b3)


def _fp2_body(f3_ref, f2_ref, w1i_ref, w1f_ref, b1_ref, w2_ref, b2_ref, out_ref):
    c3 = _nt(w1i_ref[...], f3_ref[0])
    h = c3 + _nt(w1f_ref[...], f2_ref[0])
    h = jax.nn.relu((h + b1_ref[...]) * BN_SCALE)
    h = jnp.dot(w2_ref[...], h, preferred_element_type=jnp.float32)
    out_ref[0] = jax.nn.relu((h + b2_ref[...]) * BN_SCALE)


def _fp2(f3, f2, w1i, w1f, b1col, w2, b2col):
    return pl.pallas_call(
        _fp2_body,
        grid=(B,),
        in_specs=[
            pl.BlockSpec((1, 1, 1024), lambda b: (b, 0, 0)),
            pl.BlockSpec((1, S2, 256), lambda b: (b, 0, 0)),
            pl.BlockSpec((1024, 1024), lambda b: (0, 0)),
            pl.BlockSpec((1024, 256), lambda b: (0, 0)),
            pl.BlockSpec((1024, 1), lambda b: (0, 0)),
            pl.BlockSpec((1024, 1024), lambda b: (0, 0)),
            pl.BlockSpec((1024, 1), lambda b: (0, 0)),
        ],
        out_specs=pl.BlockSpec((1, 1024, S2), lambda b: (b, 0, 0)),
        out_shape=jax.ShapeDtypeStruct((B, 1024, S2), jnp.float32),
    )(f3, f2, w1i, w1f, b1col, w2, b2col)


def _fp_body(m, kx_ref, ky_ref, kz_ref, ux_ref, uy_ref, uz_ref,
             kft_ref, uft_ref, w1i_ref, w1f_ref, b1_ref, w2_ref, b2_ref, out_ref):
    b = pl.program_id(0)
    bcol = lax.broadcasted_iota(jnp.int32, (m, B), 1)

    def col(ref):
        return jnp.sum(jnp.where(bcol == b, ref[...], 0.0), axis=1, keepdims=True)

    kx = col(kx_ref)
    ky = col(ky_ref)
    kz = col(kz_ref)
    ux = ux_ref[0]
    uy = uy_ref[0]
    uz = uz_ref[0]
    d2 = (ux - kx) ** 2 + (uy - ky) ** 2 + (uz - kz) ** 2
    sub = lax.broadcasted_iota(jnp.int32, d2.shape, 0)
    at = jnp.zeros(d2.shape, dtype=jnp.float32)
    picks = []
    dd = d2
    for _ in range(3):
        mn = jnp.min(dd, axis=0, keepdims=True)
        j = jnp.min(jnp.where(dd == mn, sub, m), axis=0, keepdims=True)
        picks.append((1.0 / (mn + 1e-8), j))
        dd = jnp.where(sub == j, jnp.float32(np.inf), dd)
    norm = picks[0][0] + picks[1][0] + picks[2][0]
    for r, j in picks:
        at = at + jnp.where(sub == j, r / norm, 0.0)
    interp = jnp.dot(kft_ref[0], at, preferred_element_type=jnp.float32)
    h = jnp.dot(w1i_ref[...], interp, preferred_element_type=jnp.float32)
    h = h + _nt(w1f_ref[...], uft_ref[0])
    h = jax.nn.relu((h + b1_ref[...]) * BN_SCALE)
    h = jnp.dot(w2_ref[...], h, preferred_element_type=jnp.float32)
    out_ref[0] = jax.nn.relu((h + b2_ref[...]) * BN_SCALE)


def _fp(kxT, kyT, kzT, ux, uy, uz, kfT, uf, w1i, w1f, b1col, w2, b2col, nchunk):
    m = kxT.shape[0]
    ntot = ux.shape[1]
    cin = kfT.shape[1]
    cf = w1f.shape[1]
    cout = w2.shape[0]
    cmid = w2.shape[1]
    grid = (B, ntot // nchunk)
    return pl.pallas_call(
        functools.partial(_fp_body, m),
        grid=grid,
        in_specs=[
            pl.BlockSpec((m, B), lambda b, j: (0, 0)),
            pl.BlockSpec((m, B), lambda b, j: (0, 0)),
            pl.BlockSpec((m, B), lambda b, j: (0, 0)),
            pl.BlockSpec((1, 1, nchunk), lambda b, j: (b, 0, j)),
            pl.BlockSpec((1, 1, nchunk), lambda b, j: (b, 0, j)),
            pl.BlockSpec((1, 1, nchunk), lambda b, j: (b, 0, j)),
            pl.BlockSpec((1, cin, m), lambda b, j: (b, 0, 0)),
            pl.BlockSpec((1, nchunk, cf), lambda b, j: (b, j, 0)),
            pl.BlockSpec((cmid, cin), lambda b, j: (0, 0)),
            pl.BlockSpec((cmid, cf), lambda b, j: (0, 0)),
            pl.BlockSpec((cmid, 1), lambda b, j: (0, 0)),
            pl.BlockSpec((cout, cmid), lambda b, j: (0, 0)),
            pl.BlockSpec((cout, 1), lambda b, j: (0, 0)),
        ],
        out_specs=pl.BlockSpec((1, cout, nchunk), lambda b, j: (b, 0, j)),
        out_shape=jax.ShapeDtypeStruct((B, cout, ntot), jnp.float32),
    )(kxT, kyT, kzT, ux[:, None, :], uy[:, None, :], uz[:, None, :],
      kfT, uf, w1i, w1f, b1col, w2, b2col)


def kernel(pointcloud, params):
    pc = pointcloud
    x = pc[:, :, 0]
    y = pc[:, :, 1]
    z = pc[:, :, 2]

    cx1, cy1, cz1 = _fps(x, y, z, S1, 16)
    cx1T = jnp.transpose(cx1)
    cy1T = jnp.transpose(cy1)
    cz1T = jnp.transpose(cz1)
    idx1 = _ball_query(x, y, z, cx1T, cy1T, cz1T, NS1, R1SQ, 256)

    cx2, cy2, cz2 = _fps(cx1, cy1, cz1, S2, 4)
    cx2T = jnp.transpose(cx2)
    cy2T = jnp.transpose(cy2)
    cz2T = jnp.transpose(cz2)
    idx2 = _ball_query(cx1, cy1, cz1, cx2T, cy2T, cz2T, NS2, R2SQ, S2)

    table1 = jnp.pad(pc, ((0, 0), (0, 0), (0, 10))).reshape(B * N, 16)
    boff1 = (jnp.arange(B, dtype=jnp.int32) * N)[:, None, None]
    flat1 = jnp.transpose(idx1 + boff1, (0, 2, 1)).reshape(-1)
    g1 = _gather_rows(table1, flat1, 16).reshape(B, NS1 * S1, 16)

    c1rows = jnp.stack([cx1, cy1, cz1], axis=2)

    p = params["sa1"]
    w1t = jnp.transpose(p[0]["W"])
    w1p = jnp.pad(w1t, ((0, 10), (0, 0)))
    f1 = _sa_mlp_max(
        [(g1, w1p)], c1rows, w1t[0:3], p[0]["b"][None],
        jnp.transpose(p[1]["W"]), p[1]["b"][None],
        jnp.transpose(p[2]["W"]), p[2]["b"][None], S1, NS1)

    boff2 = (jnp.arange(B, dtype=jnp.int32) * S1)[:, None, None]
    flat2 = jnp.transpose(idx2 + boff2, (0, 2, 1)).reshape(-1)
    table2x = jnp.pad(c1rows, ((0, 0), (0, 0), (0, 13))).reshape(B * S1, 16)
    g2x = _gather_rows(table2x, flat2, 16).reshape(B, NS2 * S2, 16)
    table2f = f1.reshape(B * S1, 128)
    g2f = _gather_rows(table2f, flat2, 128).reshape(B, NS2 * S2, 128)

    c2rows = jnp.stack([cx2, cy2, cz2], axis=2)

    p = params["sa2"]
    w1t2 = jnp.transpose(p[0]["W"])
    w1x2p = jnp.pad(w1t2[0:3], ((0, 13), (0, 0)))
    f2 = _sa_mlp_max(
        [(g2x, w1x2p), (g2f, w1t2[3:131])], c2rows, w1t2[0:3], p[0]["b"][None],
        jnp.transpose(p[1]["W"]), p[1]["b"][None],
        jnp.transpose(p[2]["W"]), p[2]["b"][None], S2, NS2)

    p = params["sa3"]
    f3 = _sa3(
        c2rows, f2, jnp.transpose(p[0]["W"])[0:3],
        jnp.transpose(p[0]["W"])[3:259], p[0]["b"][None],
        jnp.transpose(p[1]["W"]), p[1]["b"][None],
        jnp.transpose(p[2]["W"]), p[2]["b"][None])

    p = params["fp2"]
    fp2_out = _fp2(
        f3, f2,
        p[0]["W"][:, :1024], p[0]["W"][:, 1024:], p[0]["b"][:, None],
        p[1]["W"], p[1]["b"][:, None])

    p = params["fp1"]
    fp1_out = _fp(
        cx2T, cy2T, cz2T, cx1, cy1, cz1, fp2_out, f1,
        p[0]["W"][:, :1024], p[0]["W"][:, 1024:], p[0]["b"][:, None],
        p[1]["W"], p[1]["b"][:, None], S1)

    p = params["fp0"]
    w1f0 = jnp.concatenate(
        [jnp.zeros((256, 3), jnp.float32), p[0]["W"][:, 1024:]], axis=1)
    out = _fp(
        cx1T, cy1T, cz1T, x, y, z, fp1_out, pc,
        p[0]["W"][:, :1024], w1f0, p[0]["b"][:, None],
        p[1]["W"], p[1]["b"][:, None], 2048)
    return out


# speedup vs baseline: 1.1063x; 1.0330x over previous
import functools

import jax
import jax.numpy as jnp
import numpy as np
from jax import lax
from jax.experimental import pallas as pl
from jax.experimental.pallas import tpu as pltpu
from jax.experimental.pallas import tpu_sc as plsc

B = 4
N = 8192
S1 = 512
NS1 = 32
S2 = 128
NS2 = 64
R1SQ = 0.2 ** 2
R2SQ = 0.4 ** 2
BN_SCALE = 1.0 / np.sqrt(1.0 + 1e-5)


def _nt(a, b):
    return lax.dot_general(a, b, (((1,), (1,)), ((), ())),
                           preferred_element_type=jnp.float32)


def _fps_body(npoint, rows, cols, x_ref, y_ref, z_ref, cx_ref, cy_ref, cz_ref):
    n = rows * cols
    x = x_ref[...]
    y = y_ref[...]
    z = z_ref[...]
    gidx = (lax.broadcasted_iota(jnp.int32, (B, rows, cols), 1) * cols
            + lax.broadcasted_iota(jnp.int32, (B, rows, cols), 2))
    lane_s = lax.broadcasted_iota(jnp.int32, (B, npoint), 1)

    def rmin(a):
        return jnp.min(jnp.min(a, axis=2, keepdims=True), axis=1, keepdims=True)

    def rmax(a):
        return jnp.max(jnp.max(a, axis=2, keepdims=True), axis=1, keepdims=True)

    def rsum(a):
        return jnp.sum(jnp.sum(a, axis=2, keepdims=True), axis=1, keepdims=True)

    def body(i, state):
        dists, far, ax, ay, az = state
        sel = gidx == far
        cx = rsum(jnp.where(sel, x, 0.0))
        cy = rsum(jnp.where(sel, y, 0.0))
        cz = rsum(jnp.where(sel, z, 0.0))
        hit = lane_s == i
        ax = jnp.where(hit, cx[:, 0], ax)
        ay = jnp.where(hit, cy[:, 0], ay)
        az = jnp.where(hit, cz[:, 0], az)
        d = (x - cx) ** 2 + (y - cy) ** 2 + (z - cz) ** 2
        dists = jnp.minimum(dists, d)
        m = rmax(dists)
        far = rmin(jnp.where(dists == m, gidx, n))
        return dists, far, ax, ay, az

    dists0 = jnp.full((B, rows, cols), 1e10, dtype=jnp.float32)
    far0 = jnp.zeros((B, 1, 1), dtype=jnp.int32)
    acc0 = jnp.zeros((B, npoint), dtype=jnp.float32)
    _, _, ax, ay, az = lax.fori_loop(0, npoint, body, (dists0, far0, acc0, acc0, acc0))
    cx_ref[...] = ax
    cy_ref[...] = ay
    cz_ref[...] = az


def _fps(x, y, z, npoint, rows):
    n = x.shape[1]
    cols = n // rows
    out = jax.ShapeDtypeStruct((B, npoint), jnp.float32)
    r3 = lambda a: a.reshape(B, rows, cols)
    return pl.pallas_call(
        functools.partial(_fps_body, npoint, rows, cols),
        out_shape=(out, out, out),
    )(r3(x), r3(y), r3(z))


def _bq_body(n, nsample, r2, sc, x_ref, y_ref, z_ref, cx_ref, cy_ref, cz_ref, out_ref):
    b = pl.program_id(0)
    x = x_ref[0]
    y = y_ref[0]
    z = z_ref[0]
    bcol = lax.broadcasted_iota(jnp.int32, (sc, B), 1)

    def col(ref):
        return jnp.sum(jnp.where(bcol == b, ref[...], 0.0), axis=1, keepdims=True)

    cx = col(cx_ref)
    cy = col(cy_ref)
    cz = col(cz_ref)
    d = (cx - x) ** 2 + (cy - y) ** 2 + (cz - z) ** 2
    mask = d < r2
    mi = mask.astype(jnp.int32)
    s = mi
    sh = 1
    while sh < n:
        zpad = jnp.zeros((sc, sh), dtype=jnp.int32)
        s = s + jnp.concatenate([zpad, s[:, : n - sh]], axis=1)
        sh *= 2
    count = s[:, n - 1:]
    cols = []
    for t in range(nsample // 2):
        e = jnp.where(s <= 2 * t, 16385, jnp.where(s <= 2 * t + 1, 16384, 0))
        v = jnp.sum(e, axis=1, keepdims=True)
        cols.append(jnp.bitwise_and(v, 16383))
        cols.append(jnp.right_shift(v, 14))
    vals = jnp.concatenate(cols, axis=1)
    first = vals[:, 0:1]
    karr = lax.broadcasted_iota(jnp.int32, (sc, nsample), 1)
    out_ref[0] = jnp.where(karr < count, vals, first)


def _ball_query(x, y, z, cxT, cyT, czT, nsample, r2, sc):
    n = x.shape[1]
    s_tot = cxT.shape[0]
    grid = (B, s_tot // sc)
    return pl.pallas_call(
        functools.partial(_bq_body, n, nsample, r2, sc),
        grid=grid,
        in_specs=[
            pl.BlockSpec((1, 1, n), lambda b, j: (b, 0, 0)),
            pl.BlockSpec((1, 1, n), lambda b, j: (b, 0, 0)),
            pl.BlockSpec((1, 1, n), lambda b, j: (b, 0, 0)),
            pl.BlockSpec((sc, B), lambda b, j: (j, 0)),
            pl.BlockSpec((sc, B), lambda b, j: (j, 0)),
            pl.BlockSpec((sc, B), lambda b, j: (j, 0)),
        ],
        out_specs=pl.BlockSpec((1, sc, nsample), lambda b, j: (b, j, 0)),
        out_shape=jax.ShapeDtypeStruct((B, s_tot, nsample), jnp.int32),
    )(x[:, None, :], y[:, None, :], z[:, None, :], cxT, cyT, czT)


def _gather_rows(table, idx, D):
    ntot = idx.shape[0]
    NW = 32
    per_w = ntot // NW
    nchunk = per_w // 128
    mesh = plsc.VectorSubcoreMesh(core_axis_name="c", subcore_axis_name="s")
    idx2d = idx.reshape(NW * nchunk, 128)
    small = per_w * D * 4 <= 400 * 1024

    if small:
        scratch = [
            pltpu.VMEM((nchunk, 128), jnp.int32),
            pltpu.VMEM((per_w, D), jnp.float32),
            pltpu.SemaphoreType.DMA,
        ]

        def body(table_hbm, idx_hbm, out_hbm, idx_v, rows_v, sem):
            wid = lax.axis_index("s") * 2 + lax.axis_index("c")
            pltpu.sync_copy(idx_hbm.at[pl.ds(wid * nchunk, nchunk)], idx_v)
            copies = [
                pltpu.async_copy(
                    table_hbm.at[idx_v.at[j]],
                    rows_v.at[pl.ds(j * 128, 128)], sem)
                for j in range(nchunk)
            ]
            for c in copies:
                c.wait()
            pltpu.sync_copy(rows_v, out_hbm.at[pl.ds(wid * per_w, per_w)])
    else:
        scratch = [
            pltpu.VMEM((nchunk, 128), jnp.int32),
            pltpu.VMEM((2, 128, D), jnp.float32),
            pltpu.SemaphoreType.DMA,
            pltpu.SemaphoreType.DMA,
        ]

        def body(table_hbm, idx_hbm, out_hbm, idx_v, rows_v, sem0, sem1):
            wid = lax.axis_index("s") * 2 + lax.axis_index("c")
            sems = (sem0, sem1)
            pltpu.sync_copy(idx_hbm.at[pl.ds(wid * nchunk, nchunk)], idx_v)
            pending = pltpu.async_copy(
                table_hbm.at[idx_v.at[0]], rows_v.at[0], sems[0])
            for j in range(nchunk):
                if j + 1 < nchunk:
                    nxt = pltpu.async_copy(
                        table_hbm.at[idx_v.at[j + 1]],
                        rows_v.at[(j + 1) % 2], sems[(j + 1) % 2])
                pending.wait()
                pltpu.sync_copy(
                    rows_v.at[j % 2],
                    out_hbm.at[pl.ds(wid * per_w + j * 128, 128)])
                if j + 1 < nchunk:
                    pending = nxt

    k = pl.kernel(
        body,
        mesh=mesh,
        out_type=jax.ShapeDtypeStruct((ntot, D), jnp.float32),
        scratch_types=scratch,
        compiler_params=pltpu.CompilerParams(use_tc_tiling_on_sc=False),
    )
    return k(table, idx2d)


def _sa_body(S, ns, ng, *refs):
    g_refs = refs[:ng]
    w1_refs = refs[ng:2 * ng]
    (c_ref, w1x_ref, b1_ref, w2_ref, b2_ref, w3_ref, b3_ref, out_ref) = refs[2 * ng:]
    h = jnp.dot(g_refs[0][0], w1_refs[0][...], preferred_element_type=jnp.float32)
    for i in range(1, ng):
        h = h + jnp.dot(g_refs[i][0], w1_refs[i][...],
                        preferred_element_type=jnp.float32)
    o = jnp.dot(c_ref[0], w1x_ref[...], preferred_element_type=jnp.float32)
    h = jnp.concatenate(
        [h[k * S:(k + 1) * S] - o for k in range(ns)], axis=0)
    h = jax.nn.relu((h + b1_ref[...]) * BN_SCALE)
    h = jnp.dot(h, w2_ref[...], preferred_element_type=jnp.float32)
    h = jax.nn.relu((h + b2_ref[...]) * BN_SCALE)
    h = jnp.dot(h, w3_ref[...], preferred_element_type=jnp.float32)
    h = jax.nn.relu((h + b3_ref[...]) * BN_SCALE)
    m = h[0:S]
    for k in range(1, ns):
        m = jnp.maximum(m, h[k * S:(k + 1) * S])
    out_ref[0] = m


def _sa_mlp_max(gws, crows, w1x, b1, w2, b2, w3, b3, S, ns):
    ng = len(gws)
    cout = w3.shape[1]
    c1 = w2.shape[0]
    c2 = w2.shape[1]
    gspecs = [pl.BlockSpec((1, S * ns, g.shape[2]), lambda b: (b, 0, 0))
              for g, _ in gws]
    wspecs = [pl.BlockSpec((w.shape[0], c1), lambda b: (0, 0)) for _, w in gws]
    return pl.pallas_call(
        functools.partial(_sa_body, S, ns, ng),
        grid=(B,),
        in_specs=gspecs + wspecs + [
            pl.BlockSpec((1, S, 3), lambda b: (b, 0, 0)),
            pl.BlockSpec((3, c1), lambda b: (0, 0)),
            pl.BlockSpec((1, c1), lambda b: (0, 0)),
            pl.BlockSpec((c1, c2), lambda b: (0, 0)),
            pl.BlockSpec((1, c2), lambda b: (0, 0)),
            pl.BlockSpec((c2, cout), lambda b: (0, 0)),
            pl.BlockSpec((1, cout), lambda b: (0, 0)),
        ],
        out_specs=pl.BlockSpec((1, S, cout), lambda b: (b, 0, 0)),
        out_shape=jax.ShapeDtypeStruct((B, S, cout), jnp.float32),
    )(*[g for g, _ in gws], *[w for _, w in gws],
      crows, w1x, b1, w2, b2, w3, b3)


def _sa3_body(xyz_ref, f2_ref, w1x_ref, w1f_ref, b1_ref, w2_ref, b2_ref,
              w3_ref, b3_ref, out_ref):
    h = jnp.dot(xyz_ref[0], w1x_ref[...], preferred_element_type=jnp.float32)
    h = h + jnp.dot(f2_ref[0], w1f_ref[...], preferred_element_type=jnp.float32)
    h = jax.nn.relu((h + b1_ref[...]) * BN_SCALE)
    h = jnp.dot(h, w2_ref[...], preferred_element_type=jnp.float32)
    h = jax.nn.relu((h + b2_ref[...]) * BN_SCALE)
    h = jnp.dot(h, w3_ref[...], preferred_element_type=jnp.float32)
    h = jax.nn.relu((h + b3_ref[...]) * BN_SCALE)
    out_ref[0] = jnp.max(h, axis=0, keepdims=True)


def _sa3(xyz2_rows, f2, w1x, w1f, b1, w2, b2, w3, b3):
    return pl.pallas_call(
        _sa3_body,
        grid=(B,),
        in_specs=[
            pl.BlockSpec((1, S2, 3), lambda b: (b, 0, 0)),
            pl.BlockSpec((1, S2, 256), lambda b: (b, 0, 0)),
            pl.BlockSpec((3, 256), lambda b: (0, 0)),
            pl.BlockSpec((256, 256), lambda b: (0, 0)),
            pl.BlockSpec((1, 256), lambda b: (0, 0)),
            pl.BlockSpec((256, 512), lambda b: (0, 0)),
            pl.BlockSpec((1, 512), lambda b: (0, 0)),
            pl.BlockSpec((512, 1024), lambda b: (0, 0)),
            pl.BlockSpec((1, 1024), lambda b: (0, 0)),
        ],
        out_specs=pl.BlockSpec((1, 1, 1024), lambda b: (b, 0, 0)),
        out_shape=jax.ShapeDtypeStruct((B, 1, 1024), jnp.float32),
    )(xyz2_rows, f2, w1x, w1f, b1, w2, b2, w3, b3)


def _fp2_body(f3_ref, f2_ref, w1i_ref, w1f_ref, b1_ref, w2_ref, b2_ref, out_ref):
    c3 = _nt(w1i_ref[...], f3_ref[0])
    h = c3 + _nt(w1f_ref[...], f2_ref[0])
    h = jax.nn.relu((h + b1_ref[...]) * BN_SCALE)
    h = jnp.dot(w2_ref[...], h, preferred_element_type=jnp.float32)
    out_ref[0] = jax.nn.relu((h + b2_ref[...]) * BN_SCALE)


def _fp2(f3, f2, w1i, w1f, b1col, w2, b2col):
    return pl.pallas_call(
        _fp2_body,
        grid=(B,),
        in_specs=[
            pl.BlockSpec((1, 1, 1024), lambda b: (b, 0, 0)),
            pl.BlockSpec((1, S2, 256), lambda b: (b, 0, 0)),
            pl.BlockSpec((1024, 1024), lambda b: (0, 0)),
            pl.BlockSpec((1024, 256), lambda b: (0, 0)),
            pl.BlockSpec((1024, 1), lambda b: (0, 0)),
            pl.BlockSpec((1024, 1024), lambda b: (0, 0)),
            pl.BlockSpec((1024, 1), lambda b: (0, 0)),
        ],
        out_specs=pl.BlockSpec((1, 1024, S2), lambda b: (b, 0, 0)),
        out_shape=jax.ShapeDtypeStruct((B, 1024, S2), jnp.float32),
    )(f3, f2, w1i, w1f, b1col, w2, b2col)


def _fp_body(m, kx_ref, ky_ref, kz_ref, ux_ref, uy_ref, uz_ref,
             kft_ref, uft_ref, w1i_ref, w1f_ref, b1_ref, w2_ref, b2_ref, out_ref):
    b = pl.program_id(0)
    bcol = lax.broadcasted_iota(jnp.int32, (m, B), 1)

    def col(ref):
        return jnp.sum(jnp.where(bcol == b, ref[...], 0.0), axis=1, keepdims=True)

    kx = col(kx_ref)
    ky = col(ky_ref)
    kz = col(kz_ref)
    ux = ux_ref[0]
    uy = uy_ref[0]
    uz = uz_ref[0]
    d2 = (ux - kx) ** 2 + (uy - ky) ** 2 + (uz - kz) ** 2
    sub = lax.broadcasted_iota(jnp.int32, d2.shape, 0)
    at = jnp.zeros(d2.shape, dtype=jnp.float32)
    picks = []
    dd = d2
    for _ in range(3):
        mn = jnp.min(dd, axis=0, keepdims=True)
        j = jnp.min(jnp.where(dd == mn, sub, m), axis=0, keepdims=True)
        picks.append((1.0 / (mn + 1e-8), j))
        dd = jnp.where(sub == j, jnp.float32(np.inf), dd)
    norm = picks[0][0] + picks[1][0] + picks[2][0]
    for r, j in picks:
        at = at + jnp.where(sub == j, r / norm, 0.0)
    interp = jnp.dot(kft_ref[0], at, preferred_element_type=jnp.float32)
    h = jnp.dot(w1i_ref[...], interp, preferred_element_type=jnp.float32)
    h = h + _nt(w1f_ref[...], uft_ref[0])
    h = jax.nn.relu((h + b1_ref[...]) * BN_SCALE)
    h = jnp.dot(w2_ref[...], h, preferred_element_type=jnp.float32)
    out_ref[0] = jax.nn.relu((h + b2_ref[...]) * BN_SCALE)


def _fp(kxT, kyT, kzT, ux, uy, uz, kfT, uf, w1i, w1f, b1col, w2, b2col, nchunk):
    m = kxT.shape[0]
    ntot = ux.shape[1]
    cin = kfT.shape[1]
    cf = w1f.shape[1]
    cout = w2.shape[0]
    cmid = w2.shape[1]
    grid = (B, ntot // nchunk)
    return pl.pallas_call(
        functools.partial(_fp_body, m),
        grid=grid,
        in_specs=[
            pl.BlockSpec((m, B), lambda b, j: (0, 0)),
            pl.BlockSpec((m, B), lambda b, j: (0, 0)),
            pl.BlockSpec((m, B), lambda b, j: (0, 0)),
            pl.BlockSpec((1, 1, nchunk), lambda b, j: (b, 0, j)),
            pl.BlockSpec((1, 1, nchunk), lambda b, j: (b, 0, j)),
            pl.BlockSpec((1, 1, nchunk), lambda b, j: (b, 0, j)),
            pl.BlockSpec((1, cin, m), lambda b, j: (b, 0, 0)),
            pl.BlockSpec((1, nchunk, cf), lambda b, j: (b, j, 0)),
            pl.BlockSpec((cmid, cin), lambda b, j: (0, 0)),
            pl.BlockSpec((cmid, cf), lambda b, j: (0, 0)),
            pl.BlockSpec((cmid, 1), lambda b, j: (0, 0)),
            pl.BlockSpec((cout, cmid), lambda b, j: (0, 0)),
            pl.BlockSpec((cout, 1), lambda b, j: (0, 0)),
        ],
        out_specs=pl.BlockSpec((1, cout, nchunk), lambda b, j: (b, 0, j)),
        out_shape=jax.ShapeDtypeStruct((B, cout, ntot), jnp.float32),
    )(kxT, kyT, kzT, ux[:, None, :], uy[:, None, :], uz[:, None, :],
      kfT, uf, w1i, w1f, b1col, w2, b2col)


def kernel(pointcloud, params):
    pc = pointcloud
    x = pc[:, :, 0]
    y = pc[:, :, 1]
    z = pc[:, :, 2]

    cx1, cy1, cz1 = _fps(x, y, z, S1, 16)
    cx1T = jnp.transpose(cx1)
    cy1T = jnp.transpose(cy1)
    cz1T = jnp.transpose(cz1)
    idx1 = _ball_query(x, y, z, cx1T, cy1T, cz1T, NS1, R1SQ, 512)

    cx2, cy2, cz2 = _fps(cx1, cy1, cz1, S2, 4)
    cx2T = jnp.transpose(cx2)
    cy2T = jnp.transpose(cy2)
    cz2T = jnp.transpose(cz2)
    idx2 = _ball_query(cx1, cy1, cz1, cx2T, cy2T, cz2T, NS2, R2SQ, S2)

    table1 = jnp.pad(pc, ((0, 0), (0, 0), (0, 10))).reshape(B * N, 16)
    boff1 = (jnp.arange(B, dtype=jnp.int32) * N)[:, None, None]
    flat1 = jnp.transpose(idx1 + boff1, (0, 2, 1)).reshape(-1)
    g1 = _gather_rows(table1, flat1, 16).reshape(B, NS1 * S1, 16)

    c1rows = jnp.stack([cx1, cy1, cz1], axis=2)

    p = params["sa1"]
    w1t = jnp.transpose(p[0]["W"])
    w1p = jnp.pad(w1t, ((0, 10), (0, 0)))
    f1 = _sa_mlp_max(
        [(g1, w1p)], c1rows, w1t[0:3], p[0]["b"][None],
        jnp.transpose(p[1]["W"]), p[1]["b"][None],
        jnp.transpose(p[2]["W"]), p[2]["b"][None], S1, NS1)

    boff2 = (jnp.arange(B, dtype=jnp.int32) * S1)[:, None, None]
    flat2 = jnp.transpose(idx2 + boff2, (0, 2, 1)).reshape(-1)
    table2x = jnp.pad(c1rows, ((0, 0), (0, 0), (0, 13))).reshape(B * S1, 16)
    g2x = _gather_rows(table2x, flat2, 16).reshape(B, NS2 * S2, 16)
    table2f = f1.reshape(B * S1, 128)
    g2f = _gather_rows(table2f, flat2, 128).reshape(B, NS2 * S2, 128)

    c2rows = jnp.stack([cx2, cy2, cz2], axis=2)

    p = params["sa2"]
    w1t2 = jnp.transpose(p[0]["W"])
    w1x2p = jnp.pad(w1t2[0:3], ((0, 13), (0, 0)))
    f2 = _sa_mlp_max(
        [(g2x, w1x2p), (g2f, w1t2[3:131])], c2rows, w1t2[0:3], p[0]["b"][None],
        jnp.transpose(p[1]["W"]), p[1]["b"][None],
        jnp.transpose(p[2]["W"]), p[2]["b"][None], S2, NS2)

    p = params["sa3"]
    f3 = _sa3(
        c2rows, f2, jnp.transpose(p[0]["W"])[0:3],
        jnp.transpose(p[0]["W"])[3:259], p[0]["b"][None],
        jnp.transpose(p[1]["W"]), p[1]["b"][None],
        jnp.transpose(p[2]["W"]), p[2]["b"][None])

    p = params["fp2"]
    fp2_out = _fp2(
        f3, f2,
        p[0]["W"][:, :1024], p[0]["W"][:, 1024:], p[0]["b"][:, None],
        p[1]["W"], p[1]["b"][:, None])

    p = params["fp1"]
    fp1_out = _fp(
        cx2T, cy2T, cz2T, cx1, cy1, cz1, fp2_out, f1,
        p[0]["W"][:, :1024], p[0]["W"][:, 1024:], p[0]["b"][:, None],
        p[1]["W"], p[1]["b"][:, None], S1)

    p = params["fp0"]
    w1f0 = jnp.concatenate(
        [jnp.zeros((256, 3), jnp.float32), p[0]["W"][:, 1024:]], axis=1)
    out = _fp(
        cx1T, cy1T, cz1T, x, y, z, fp1_out, pc,
        p[0]["W"][:, :1024], w1f0, p[0]["b"][:, None],
        p[1]["W"], p[1]["b"][:, None], 2048)
    return out
